# baseline (speedup 1.0000x reference)
import jax
import jax.numpy as jnp
from jax import lax
from jax.experimental import pallas as pl

_R, _C = 128, 100000
_BR = 8
_NBLK = _R // _BR


def _tc_body(l_ref, o_ref, alp_ref, ent_ref):
    l = l_ref[...]
    o = o_ref[...]
    s = jnp.sum(l, axis=1, keepdims=True)
    sll = jnp.sum(l * jnp.log(l), axis=1, keepdims=True)
    m = jnp.max(o, axis=1, keepdims=True)
    iota = lax.broadcasted_iota(jnp.int32, (_BR, _C), 1)
    big = jnp.int32(2**30)
    idx = jnp.min(jnp.where(o == m, iota, big), axis=1, keepdims=True)
    lsel = jnp.sum(jnp.where(iota == idx, l, 0.0), axis=1, keepdims=True)
    logs = jnp.log(s)
    alp = jnp.log(lsel) - logs
    ent = logs - sll / s
    alp_ref[...] = jnp.broadcast_to(alp, (_BR, 128))
    ent_ref[...] = jnp.broadcast_to(ent, (_BR, 128))


def kernel(logits, outputs):
    alp, ent = pl.pallas_call(
        _tc_body,
        grid=(_NBLK,),
        in_specs=[
            pl.BlockSpec((_BR, _C), lambda i: (i, 0)),
            pl.BlockSpec((_BR, _C), lambda i: (i, 0)),
        ],
        out_specs=[
            pl.BlockSpec((_BR, 128), lambda i: (i, 0)),
            pl.BlockSpec((_BR, 128), lambda i: (i, 0)),
        ],
        out_shape=[
            jax.ShapeDtypeStruct((_R, 128), jnp.float32),
            jax.ShapeDtypeStruct((_R, 128), jnp.float32),
        ],
    )(logits, outputs)
    return (alp[:, 0], ent[:, 0])
